# Initial kernel scaffold; baseline (speedup 1.0000x reference)
import jax
import jax.numpy as jnp
from jax.experimental import pallas as pl


def kernel(x, edge_index, W1_0, b1_0, W2_0, b2_0, W1_1, b1_1, W2_1, b2_1):
    raise NotImplementedError("write your pallas kernel here")


# speedup vs baseline: 2.7452x; 2.7452x over previous
import functools

import jax
import jax.numpy as jnp
from jax import lax
from jax.experimental import pallas as pl
from jax.experimental.pallas import tpu as pltpu
from jax.experimental.pallas import tpu_sc as plsc

N = 10000
E = 320000
NHID = 128
NCORE = 2
NSUB = 16
NW = NCORE * NSUB
CHUNK = 128
CHUNKS = 80
PT = CHUNK * CHUNKS
E_PAD = PT * NW
ROWS_PER_TILE = 640
ACC_ROWS = ROWS_PER_TILE * NSUB

_sc_mesh = plsc.VectorSubcoreMesh(core_axis_name="c", subcore_axis_name="s")


@functools.partial(
    pl.kernel,
    out_type=(
        jax.ShapeDtypeStruct((ACC_ROWS, NHID), jnp.float32),
        jax.ShapeDtypeStruct((ACC_ROWS, NHID), jnp.float32),
    ),
    mesh=_sc_mesh,
    scratch_types=[
        pltpu.VMEM((CHUNKS, CHUNK), jnp.int32),
        pltpu.VMEM((CHUNKS, CHUNK), jnp.int32),
        pltpu.VMEM((CHUNK, NHID), jnp.float32),
        pltpu.VMEM_SHARED((ACC_ROWS, NHID), jnp.float32),
        pltpu.SemaphoreType.DMA,
    ],
)
def _sc_aggregate(h_hbm, src_hbm, dst_hbm, out0_hbm, out1_hbm,
                  src_v, dst_v, buf_v, acc_sh, sem):
    cid = lax.axis_index("c")
    sid = lax.axis_index("s")
    wid = cid * NSUB + sid
    base = sid * ROWS_PER_TILE

    zeros = jnp.zeros((16,), jnp.float32)

    def _zero_row(r, carry):
        for c in range(NHID // 16):
            buf_v[r, pl.ds(c * 16, 16)] = zeros
        return carry

    lax.fori_loop(0, CHUNK, _zero_row, 0)
    for k in range(ROWS_PER_TILE // CHUNK):
        pltpu.sync_copy(buf_v, acc_sh.at[pl.ds(base + k * CHUNK, CHUNK)])

    pltpu.sync_copy(src_hbm.at[wid], src_v)
    pltpu.sync_copy(dst_hbm.at[wid], dst_v)

    plsc.subcore_barrier()

    def _chunk(j, carry):
        pltpu.async_copy(h_hbm.at[src_v.at[j]], buf_v, sem).wait()
        pltpu.sync_copy(buf_v, acc_sh.at[dst_v.at[j]], add=True)
        return carry

    lax.fori_loop(0, CHUNKS, _chunk, 0)

    plsc.subcore_barrier()

    @pl.when(cid == 0)
    def _():
        pltpu.sync_copy(acc_sh.at[pl.ds(base, ROWS_PER_TILE)],
                        out0_hbm.at[pl.ds(base, ROWS_PER_TILE)])

    @pl.when(cid == 1)
    def _():
        pltpu.sync_copy(acc_sh.at[pl.ds(base, ROWS_PER_TILE)],
                        out1_hbm.at[pl.ds(base, ROWS_PER_TILE)])


def _mlp_body(x_ref, a0_ref, a1_ref, w1_ref, b1_ref, w2_ref, b2_ref, o_ref):
    h = x_ref[...] + a0_ref[...] + a1_ref[...]
    y = jnp.dot(h, w1_ref[...], preferred_element_type=jnp.float32) + b1_ref[...]
    y = jnp.maximum(y, 0.0)
    z = jnp.dot(y, w2_ref[...], preferred_element_type=jnp.float32) + b2_ref[...]
    o_ref[...] = jnp.maximum(z, 0.0)


_BLK = 1000
_mlp = pl.pallas_call(
    _mlp_body,
    grid=(N // _BLK,),
    in_specs=[
        pl.BlockSpec((_BLK, NHID), lambda i: (i, 0)),
        pl.BlockSpec((_BLK, NHID), lambda i: (i, 0)),
        pl.BlockSpec((_BLK, NHID), lambda i: (i, 0)),
        pl.BlockSpec((NHID, NHID), lambda i: (0, 0)),
        pl.BlockSpec((1, NHID), lambda i: (0, 0)),
        pl.BlockSpec((NHID, NHID), lambda i: (0, 0)),
        pl.BlockSpec((1, NHID), lambda i: (0, 0)),
    ],
    out_specs=pl.BlockSpec((_BLK, NHID), lambda i: (i, 0)),
    out_shape=jax.ShapeDtypeStruct((N, NHID), jnp.float32),
)


def kernel(x, edge_index, W1_0, b1_0, W2_0, b2_0, W1_1, b1_1, W2_1, b2_1):
    src = edge_index[0]
    dst = edge_index[1]
    pad = E_PAD - E
    src_p = jnp.concatenate([src, jnp.zeros((pad,), jnp.int32)])
    dst_p = jnp.concatenate([dst, jnp.full((pad,), ACC_ROWS - 1, jnp.int32)])
    src3 = src_p.reshape(NW, CHUNKS, CHUNK)
    dst3 = dst_p.reshape(NW, CHUNKS, CHUNK)

    h = x
    for (W1, b1, W2, b2) in ((W1_0, b1_0, W2_0, b2_0), (W1_1, b1_1, W2_1, b2_1)):
        a0, a1 = _sc_aggregate(h, src3, dst3)
        h = _mlp(h, a0, a1, W1, b1.reshape(1, NHID), W2, b2.reshape(1, NHID))
    return h


# speedup vs baseline: 3.4959x; 1.2735x over previous
import functools

import jax
import jax.numpy as jnp
from jax import lax
from jax.experimental import pallas as pl
from jax.experimental.pallas import tpu as pltpu
from jax.experimental.pallas import tpu_sc as plsc

N = 10000
E = 320000
NHID = 128
NCORE = 2
NSUB = 16
CHUNK = 128
TOTAL_CHUNKS = 2560
E_PAD = TOTAL_CHUNKS * CHUNK
C0 = 118
C1 = 42
ROWS_PER_TILE = 640
ACC_ROWS = ROWS_PER_TILE * NSUB

_sc_mesh = plsc.VectorSubcoreMesh(core_axis_name="c", subcore_axis_name="s")


@functools.partial(
    pl.kernel,
    out_type=(
        jax.ShapeDtypeStruct((ACC_ROWS, NHID), jnp.float32),
        jax.ShapeDtypeStruct((ACC_ROWS, NHID), jnp.float32),
    ),
    mesh=_sc_mesh,
    scratch_types=[
        pltpu.VMEM((2, CHUNK), jnp.int32),
        pltpu.VMEM((4, CHUNK), jnp.int32),
        pltpu.VMEM((CHUNK, NHID), jnp.float32),
        pltpu.VMEM((CHUNK, NHID), jnp.float32),
        pltpu.VMEM_SHARED((ACC_ROWS, NHID), jnp.float32),
        pltpu.SemaphoreType.DMA,
        pltpu.SemaphoreType.DMA,
        pltpu.SemaphoreType.DMA,
        pltpu.SemaphoreType.DMA,
        pltpu.SemaphoreType.DMA,
        pltpu.SemaphoreType.DMA,
        pltpu.SemaphoreType.DMA,
        pltpu.SemaphoreType.DMA,
        pltpu.SemaphoreType.DMA,
        pltpu.SemaphoreType.DMA,
    ],
)
def _sc_aggregate(h_hbm, src_hbm, dst_hbm, out0_hbm, out1_hbm,
                  sslab, dslab, bf0, bf1, acc_sh,
                  g0, g1, s0, s1, is0, is1, id0, id1, id2, id3):
    cid = lax.axis_index("c")
    sid = lax.axis_index("s")
    base = sid * ROWS_PER_TILE
    bufs = (bf0, bf1)
    gsem = (g0, g1)
    ssem = (s0, s1)
    issem = (is0, is1)
    idsem = (id0, id1, id2, id3)

    zeros = jnp.zeros((16,), jnp.float32)

    def _zero_row(r, carry):
        for c in range(NHID // 16):
            bf0[r, pl.ds(c * 16, 16)] = zeros
        return carry

    lax.fori_loop(0, CHUNK, _zero_row, 0)
    for k in range(ROWS_PER_TILE // CHUNK):
        pltpu.sync_copy(bf0, acc_sh.at[pl.ds(base + k * CHUNK, CHUNK)])

    plsc.subcore_barrier()

    def _fire_is(c, r):
        pltpu.async_copy(src_hbm.at[c], sslab.at[r], issem[r])

    def _wait_is(c, r):
        pltpu.make_async_copy(src_hbm.at[c], sslab.at[r], issem[r]).wait()

    def _fire_id(c, r):
        pltpu.async_copy(dst_hbm.at[c], dslab.at[r], idsem[r])

    def _wait_id(c, r):
        pltpu.make_async_copy(dst_hbm.at[c], dslab.at[r], idsem[r]).wait()

    def _fire_g(r, b):
        pltpu.async_copy(h_hbm.at[sslab.at[r]], bufs[b], gsem[b])

    def _wait_g(r, b):
        pltpu.make_async_copy(h_hbm.at[sslab.at[r]], bufs[b], gsem[b]).wait()

    def _fire_s(r, b):
        pltpu.async_copy(bufs[b], acc_sh.at[dslab.at[r]], ssem[b], add=True)

    def _wait_s(r, b):
        pltpu.make_async_copy(bufs[b], acc_sh.at[dslab.at[r]], ssem[b]).wait()

    def _run(nc, cbase):
        def _emit(j, m2, m4, first, fire_next_g, fire_is2, fire_id2):
            n2 = (m2 + 1) % 2
            if fire_next_g:
                _wait_is(cbase + j + 1, n2)
                if not first:
                    _wait_s((m4 + 3) % 4, n2)
                _fire_g(n2, n2)
            _wait_g(m2, m2)
            _wait_id(cbase + j, m4)
            _fire_s(m4, m2)
            if fire_is2:
                _fire_is(cbase + j + 2, m2)
            if fire_id2:
                _fire_id(cbase + j + 2, (m4 + 2) % 4)

        _fire_is(cbase + 0, 0)
        _fire_id(cbase + 0, 0)
        _fire_is(cbase + 1, 1)
        _fire_id(cbase + 1, 1)
        _wait_is(cbase + 0, 0)
        _fire_g(0, 0)

        _emit(0, 0, 0, True, True, nc > 2, nc > 2)

        steady_n = nc - 3
        loop_n = steady_n // 4
        tail = steady_n % 4

        if loop_n > 0:
            def _steady(g, carry):
                j = 1 + 4 * g
                for k in range(4):
                    _emit(j + k, (1 + k) % 2, (1 + k) % 4, False,
                          True, True, True)
                return carry

            lax.fori_loop(0, loop_n, _steady, 0)
        for t in range(tail):
            j = 1 + 4 * loop_n + t
            _emit(j, j % 2, j % 4, False, True, True, True)

        if nc >= 2:
            j = nc - 2
            _emit(j, j % 2, j % 4, False, True, False, False)
            j = nc - 1
            _emit(j, j % 2, j % 4, False, False, False, False)
        _wait_s((nc - 2) % 4, (nc - 2) % 2)
        _wait_s((nc - 1) % 4, (nc - 1) % 2)

    @pl.when(cid == 0)
    def _():
        _run(C0, sid * C0)

    @pl.when(cid == 1)
    def _():
        _run(C1, NSUB * C0 + sid * C1)

    plsc.subcore_barrier()

    @pl.when(cid == 0)
    def _():
        pltpu.sync_copy(acc_sh.at[pl.ds(base, ROWS_PER_TILE)],
                        out0_hbm.at[pl.ds(base, ROWS_PER_TILE)])

    @pl.when(cid == 1)
    def _():
        pltpu.sync_copy(acc_sh.at[pl.ds(base, ROWS_PER_TILE)],
                        out1_hbm.at[pl.ds(base, ROWS_PER_TILE)])


def _mlp_body(x_ref, a0_ref, a1_ref, w1_ref, b1_ref, w2_ref, b2_ref, o_ref):
    h = x_ref[...] + a0_ref[...] + a1_ref[...]
    y = jnp.dot(h, w1_ref[...], preferred_element_type=jnp.float32) + b1_ref[...]
    y = jnp.maximum(y, 0.0)
    z = jnp.dot(y, w2_ref[...], preferred_element_type=jnp.float32) + b2_ref[...]
    o_ref[...] = jnp.maximum(z, 0.0)


_BLK = 1000
_mlp = pl.pallas_call(
    _mlp_body,
    grid=(N // _BLK,),
    in_specs=[
        pl.BlockSpec((_BLK, NHID), lambda i: (i, 0)),
        pl.BlockSpec((_BLK, NHID), lambda i: (i, 0)),
        pl.BlockSpec((_BLK, NHID), lambda i: (i, 0)),
        pl.BlockSpec((NHID, NHID), lambda i: (0, 0)),
        pl.BlockSpec((1, NHID), lambda i: (0, 0)),
        pl.BlockSpec((NHID, NHID), lambda i: (0, 0)),
        pl.BlockSpec((1, NHID), lambda i: (0, 0)),
    ],
    out_specs=pl.BlockSpec((_BLK, NHID), lambda i: (i, 0)),
    out_shape=jax.ShapeDtypeStruct((N, NHID), jnp.float32),
)


def kernel(x, edge_index, W1_0, b1_0, W2_0, b2_0, W1_1, b1_1, W2_1, b2_1):
    src = edge_index[0]
    dst = edge_index[1]
    pad = E_PAD - E
    src_p = jnp.concatenate([src, jnp.zeros((pad,), jnp.int32)])
    dst_p = jnp.concatenate([dst, jnp.full((pad,), ACC_ROWS - 1, jnp.int32)])
    src2 = src_p.reshape(TOTAL_CHUNKS, CHUNK)
    dst2 = dst_p.reshape(TOTAL_CHUNKS, CHUNK)

    h = x
    for (W1, b1, W2, b2) in ((W1_0, b1_0, W2_0, b2_0), (W1_1, b1_1, W2_1, b2_1)):
        a0, a1 = _sc_aggregate(h, src2, dst2)
        h = _mlp(h, a0, a1, W1, b1.reshape(1, NHID), W2, b2.reshape(1, NHID))
    return h
